# Initial kernel scaffold; baseline (speedup 1.0000x reference)
import jax
import jax.numpy as jnp
from jax.experimental import pallas as pl


def kernel(sequence_output, valid_ids):
    raise NotImplementedError("write your pallas kernel here")


# speedup vs baseline: 1.5574x; 1.5574x over previous
import functools

import jax
import jax.numpy as jnp
from jax import lax
from jax.experimental import pallas as pl
from jax.experimental.pallas import tpu as pltpu
from jax.experimental.pallas import tpu_sc as plsc

B, S, D = 16, 2048, 768
L = 16
C = 64
NCHUNK = S // C
ZR = 16
WPB = 2
NVEC = S // L
DL = D // L


def _compact_body(x_hbm, valid_hbm, out_hbm, valid_v, srcidx_v, gbuf, zbuf, sem):
    cid = lax.axis_index("c")
    sid = lax.axis_index("s")
    wid = sid * 2 + cid
    b = wid // WPB
    sub = wid % WPB
    base = b * S

    pltpu.sync_copy(valid_hbm.at[b], valid_v)

    def _zrow(r, carry):
        for i in range(DL):
            zbuf[r, pl.ds(i * L, L)] = jnp.zeros((L,), jnp.float32)
        return carry

    lax.fori_loop(0, ZR, _zrow, 0)

    def _init(i, carry):
        srcidx_v[pl.ds(i * L, L)] = jnp.full((L,), base, jnp.int32)
        return carry

    lax.fori_loop(0, NVEC, _init, 0)

    def _prefix(i, carry):
        v = valid_v[pl.ds(i * L, L)]
        incl = plsc.cumsum(v)
        dst = incl - v + carry
        svals = base + i * L + lax.iota(jnp.int32, L)
        plsc.store_scatter(srcidx_v, [dst], svals, mask=(v == 1))
        return carry + jnp.sum(v)

    count = lax.fori_loop(0, NVEC, _prefix, jnp.int32(0))

    n_full = count // C
    rem = count - n_full * C

    def _chunk(i, carry):
        k = sub + WPB * i
        j0 = k * C
        is_gather = k < n_full
        is_partial = jnp.logical_and(k == n_full, rem > 0)

        @pl.when(jnp.logical_or(is_gather, is_partial))
        def _():
            cp = pltpu.async_copy(
                x_hbm.at[srcidx_v.at[pl.ds(j0, C)]], gbuf, sem
            )
            cp.wait()

            @pl.when(is_partial)
            def _():
                def _ztail(r, c2):
                    for q in range(DL):
                        gbuf[r, pl.ds(q * L, L)] = jnp.zeros((L,), jnp.float32)
                    return c2

                lax.fori_loop(rem, C, _ztail, 0)

            pltpu.sync_copy(gbuf, out_hbm.at[pl.ds(base + j0, C)])

        @pl.when(jnp.logical_not(jnp.logical_or(is_gather, is_partial)))
        def _():
            for q in range(C // ZR):
                pltpu.sync_copy(
                    zbuf, out_hbm.at[pl.ds(base + j0 + q * ZR, ZR)]
                )

        return carry

    lax.fori_loop(0, NCHUNK // WPB, _chunk, 0)


@functools.partial(jax.jit, static_argnums=())
def _compact(x_flat, valid_ids):
    mesh = plsc.VectorSubcoreMesh(core_axis_name="c", subcore_axis_name="s")
    f = pl.kernel(
        _compact_body,
        out_type=jax.ShapeDtypeStruct((B * S, D), jnp.float32),
        mesh=mesh,
        compiler_params=pltpu.CompilerParams(needs_layout_passes=False),
        scratch_types=[
            pltpu.VMEM((S,), jnp.int32),
            pltpu.VMEM((S,), jnp.int32),
            pltpu.VMEM((C, D), jnp.float32),
            pltpu.VMEM((ZR, D), jnp.float32),
            pltpu.SemaphoreType.DMA,
        ],
    )
    return f(x_flat, valid_ids)


def kernel(sequence_output, valid_ids):
    x_flat = sequence_output.reshape(B * S, D)
    out = _compact(x_flat, valid_ids)
    return out.reshape(B, S, D)


# speedup vs baseline: 1.8335x; 1.1773x over previous
import functools

import jax
import jax.numpy as jnp
from jax import lax
from jax.experimental import pallas as pl
from jax.experimental.pallas import tpu as pltpu
from jax.experimental.pallas import tpu_sc as plsc

B, S, D = 16, 2048, 768
L = 16
C = 64
NCHUNK = S // C
ZR = 32
WPB = 2
NVEC = S // L
DL = D // L
MCH = NCHUNK // WPB


def _compact_body(x_hbm, valid_hbm, out_hbm, valid_v, srcidx_v, gbuf0, gbuf1,
                  zbuf, sg0, sg1, sz):
    cid = lax.axis_index("c")
    sid = lax.axis_index("s")
    wid = sid * 2 + cid
    b = wid // WPB
    sub = wid % WPB
    base = b * S

    pltpu.sync_copy(valid_hbm.at[b], valid_v)

    def _zrow(r, carry):
        for i in range(DL):
            zbuf[r, pl.ds(i * L, L)] = jnp.zeros((L,), jnp.float32)
        return carry

    lax.fori_loop(0, ZR, _zrow, 0)

    def _init(i, carry):
        srcidx_v[pl.ds(i * L, L)] = jnp.full((L,), base, jnp.int32)
        return carry

    lax.fori_loop(0, NVEC, _init, 0)

    def _prefix(i, carry):
        v = valid_v[pl.ds(i * L, L)]
        incl = plsc.cumsum(v)
        dst = incl - v + carry
        svals = base + i * L + lax.iota(jnp.int32, L)
        plsc.store_scatter(srcidx_v, [dst], svals, mask=(v == 1))
        return carry + jnp.sum(v)

    count = lax.fori_loop(0, NVEC, _prefix, jnp.int32(0))

    n_full = count // C
    rem = count - n_full * C
    ng_all = n_full + jnp.where(rem > 0, 1, 0).astype(jnp.int32)
    mg = jnp.maximum(0, (ng_all - sub + 1) // WPB)

    def _zfire(i, carry):
        j0 = base + (sub + WPB * i) * C
        for q in range(C // ZR):
            pltpu.async_copy(zbuf, out_hbm.at[pl.ds(j0 + q * ZR, ZR)], sz)
        return carry

    lax.fori_loop(mg, MCH, _zfire, 0)

    def _start_g(i, buf, sem):
        k = sub + WPB * i
        pltpu.async_copy(x_hbm.at[srcidx_v.at[pl.ds(k * C, C)]], buf, sem)

    def _wait_g(buf, sem):
        pltpu.make_async_copy(x_hbm.at[pl.ds(0, C)], buf, sem).wait()

    def _finish_g(i, buf):
        k = sub + WPB * i

        @pl.when(jnp.logical_and(k == n_full, rem > 0))
        def _():
            def _ztail(r, c2):
                for q in range(DL):
                    buf[r, pl.ds(q * L, L)] = jnp.zeros((L,), jnp.float32)
                return c2

            lax.fori_loop(rem, C, _ztail, 0)

        pltpu.sync_copy(buf, out_hbm.at[pl.ds(base + k * C, C)])

    @pl.when(mg > 0)
    def _():
        _start_g(0, gbuf0, sg0)

    def _pair(i2, carry):
        i0 = 2 * i2
        i1 = i0 + 1

        @pl.when(i1 < mg)
        def _():
            _start_g(i1, gbuf1, sg1)

        @pl.when(i0 < mg)
        def _():
            _wait_g(gbuf0, sg0)
            _finish_g(i0, gbuf0)

        @pl.when(i1 < mg)
        def _():
            @pl.when(i1 + 1 < mg)
            def _():
                _start_g(i1 + 1, gbuf0, sg0)

            _wait_g(gbuf1, sg1)
            _finish_g(i1, gbuf1)

        return carry

    lax.fori_loop(0, MCH // 2, _pair, 0)

    def _zdrain(i, carry):
        pltpu.make_async_copy(x_hbm.at[pl.ds(0, ZR)], zbuf, sz).wait()
        return carry

    lax.fori_loop(0, (MCH - mg) * (C // ZR), _zdrain, 0)


@functools.partial(jax.jit, static_argnums=())
def _compact(x_flat, valid_ids):
    mesh = plsc.VectorSubcoreMesh(core_axis_name="c", subcore_axis_name="s")
    f = pl.kernel(
        _compact_body,
        out_type=jax.ShapeDtypeStruct((B * S, D), jnp.float32),
        mesh=mesh,
        compiler_params=pltpu.CompilerParams(needs_layout_passes=False),
        scratch_types=[
            pltpu.VMEM((S,), jnp.int32),
            pltpu.VMEM((S,), jnp.int32),
            pltpu.VMEM((C, D), jnp.float32),
            pltpu.VMEM((C, D), jnp.float32),
            pltpu.VMEM((ZR, D), jnp.float32),
            pltpu.SemaphoreType.DMA,
            pltpu.SemaphoreType.DMA,
            pltpu.SemaphoreType.DMA,
        ],
    )
    return f(x_flat, valid_ids)


def kernel(sequence_output, valid_ids):
    x_flat = sequence_output.reshape(B * S, D)
    out = _compact(x_flat, valid_ids)
    return out.reshape(B, S, D)
